# Initial kernel scaffold; baseline (speedup 1.0000x reference)
import functools

import jax
import jax.numpy as jnp
from jax import lax
from jax.experimental import pallas as pl
from jax.experimental.pallas import tpu as pltpu
from jax.experimental.pallas import tpu_sc as plsc

N_PAD = 10240
ROW_BLK = 512
SC_CORES = 2
SC_SUBCORES = 16
EDGE_BLK = 128


def _node_body(x_ref, w_ref, b_ref, out_ref):
    z = jnp.dot(x_ref[...], w_ref[...], preferred_element_type=jnp.float32,
                precision=lax.Precision.HIGHEST)
    z = jnp.maximum(z + b_ref[...], 0.0)
    out_ref[0] = z[:, :128]
    out_ref[1] = z[:, 128:]


def _node_proj(x_pad, node_w, node_b):
    grid = (N_PAD // ROW_BLK,)
    return pl.pallas_call(
        _node_body,
        grid=grid,
        in_specs=[
            pl.BlockSpec((ROW_BLK, 128), lambda i: (i, 0)),
            pl.BlockSpec((128, 256), lambda i: (0, 0)),
            pl.BlockSpec((1, 256), lambda i: (0, 0)),
        ],
        out_specs=pl.BlockSpec((2, ROW_BLK, 128), lambda i: (0, i, 0)),
        out_shape=jax.ShapeDtypeStruct((2, N_PAD, 128), jnp.float32),
    )(x_pad, node_w, node_b.reshape(1, 256))


def _gin_body(h_ref, a_ref, s_ref, w1_ref, b1_ref, w2_ref, b2_ref, out_ref):
    h = jnp.concatenate([h_ref[0], h_ref[1]], axis=1)
    agg = jnp.concatenate([a_ref[0], a_ref[1]], axis=1)
    z = h * s_ref[...] + agg
    z = jnp.dot(z, w1_ref[...], preferred_element_type=jnp.float32,
                precision=lax.Precision.HIGHEST)
    z = jnp.maximum(z + b1_ref[...], 0.0)
    z = jnp.dot(z, w2_ref[...], preferred_element_type=jnp.float32,
                precision=lax.Precision.HIGHEST)
    z = jnp.maximum(z + b2_ref[...], 0.0)
    out_ref[0] = z[:, :128]
    out_ref[1] = z[:, 128:]


def _gin_mlp(h_split, agg_split, scale_row, w1, b1, w2, b2):
    grid = (N_PAD // ROW_BLK,)
    return pl.pallas_call(
        _gin_body,
        grid=grid,
        in_specs=[
            pl.BlockSpec((2, ROW_BLK, 128), lambda i: (0, i, 0)),
            pl.BlockSpec((2, ROW_BLK, 128), lambda i: (0, i, 0)),
            pl.BlockSpec((1, 256), lambda i: (0, 0)),
            pl.BlockSpec((256, 256), lambda i: (0, 0)),
            pl.BlockSpec((1, 256), lambda i: (0, 0)),
            pl.BlockSpec((256, 256), lambda i: (0, 0)),
            pl.BlockSpec((1, 256), lambda i: (0, 0)),
        ],
        out_specs=pl.BlockSpec((2, ROW_BLK, 128), lambda i: (0, i, 0)),
        out_shape=jax.ShapeDtypeStruct((2, N_PAD, 128), jnp.float32),
    )(h_split, agg_split, scale_row, w1, b1.reshape(1, 256), w2,
      b2.reshape(1, 256))


def _pool_body(h_ref, batch_ref, w1_ref, b1_ref, w2_ref, b2_ref, out_ref,
               sums_ref, counts_ref):
    i = pl.program_id(0)
    nblk = pl.num_programs(0)

    @pl.when(i == 0)
    def _():
        sums_ref[...] = jnp.zeros_like(sums_ref)
        counts_ref[...] = jnp.zeros_like(counts_ref)

    h = jnp.concatenate([h_ref[0], h_ref[1]], axis=1)
    b = batch_ref[0, 0, :]
    gids = lax.broadcasted_iota(jnp.int32, (16, ROW_BLK), 0)
    mask = (b[None, :] == gids).astype(jnp.float32)
    sums_ref[...] += jnp.dot(mask, h, preferred_element_type=jnp.float32,
                             precision=lax.Precision.HIGHEST)
    counts_ref[...] += jnp.sum(mask, axis=1, keepdims=True)

    @pl.when(i == nblk - 1)
    def _():
        cnt = jnp.maximum(counts_ref[...][:, :1], 1.0)
        hg = sums_ref[...] / cnt
        y = jnp.dot(hg, w1_ref[...], preferred_element_type=jnp.float32,
                    precision=lax.Precision.HIGHEST)
        y = jnp.maximum(y + b1_ref[...], 0.0)
        y = jnp.dot(y, w2_ref[...], preferred_element_type=jnp.float32,
                    precision=lax.Precision.HIGHEST)
        out_ref[...] = y + b2_ref[...]


def _pool_out(h_split, batch3, out_w1, out_b1, out_w2, out_b2):
    grid = (N_PAD // ROW_BLK,)
    return pl.pallas_call(
        _pool_body,
        grid=grid,
        in_specs=[
            pl.BlockSpec((2, ROW_BLK, 128), lambda i: (0, i, 0)),
            pl.BlockSpec((1, 1, ROW_BLK), lambda i: (i, 0, 0)),
            pl.BlockSpec((256, 256), lambda i: (0, 0)),
            pl.BlockSpec((1, 256), lambda i: (0, 0)),
            pl.BlockSpec((256, 512), lambda i: (0, 0)),
            pl.BlockSpec((1, 512), lambda i: (0, 0)),
        ],
        out_specs=pl.BlockSpec((16, 512), lambda i: (0, 0)),
        out_shape=jax.ShapeDtypeStruct((16, 512), jnp.float32),
        scratch_shapes=[
            pltpu.VMEM((16, 256), jnp.float32),
            pltpu.VMEM((16, 128), jnp.float32),
        ],
    )(h_split, batch3, out_w1, out_b1.reshape(1, 256), out_w2,
      out_b2.reshape(1, 512))


def _seg_sum(h_split, src3, dst3, blocks_per_sub):
    rows_per_sub = N_PAD // SC_SUBCORES
    mesh = plsc.VectorSubcoreMesh(core_axis_name="c", subcore_axis_name="s")

    @functools.partial(
        pl.kernel,
        out_type=jax.ShapeDtypeStruct((2, N_PAD, 128), jnp.float32),
        mesh=mesh,
        scratch_types=[
            pltpu.VMEM((blocks_per_sub, EDGE_BLK), jnp.int32),
            pltpu.VMEM((blocks_per_sub, EDGE_BLK), jnp.int32),
            pltpu.VMEM((EDGE_BLK, 128), jnp.float32),
            pltpu.VMEM_SHARED((N_PAD, 128), jnp.float32),
            pltpu.SemaphoreType.DMA,
        ],
    )
    def k(h_hbm, src_hbm, dst_hbm, out_hbm, src_v, dst_v, buf, agg_sh, sem):
        c = lax.axis_index("c")
        s = lax.axis_index("s")

        @pl.loop(0, EDGE_BLK)
        def _(r):
            @pl.loop(0, 128, step=16)
            def _(cg):
                buf[r, pl.ds(cg, 16)] = jnp.zeros((16,), jnp.float32)

        @pl.loop(0, rows_per_sub // EDGE_BLK)
        def _(i):
            pltpu.sync_copy(
                buf, agg_sh.at[pl.ds(s * rows_per_sub + i * EDGE_BLK,
                                     EDGE_BLK)])

        pltpu.sync_copy(src_hbm.at[s], src_v)
        pltpu.sync_copy(dst_hbm.at[s], dst_v)
        plsc.subcore_barrier()

        @pl.loop(0, blocks_per_sub)
        def _(j):
            pltpu.async_copy(h_hbm.at[c].at[src_v.at[j]], buf, sem).wait()
            pltpu.sync_copy(buf, agg_sh.at[dst_v.at[j]], add=True)

        plsc.subcore_barrier()

        base = s * rows_per_sub
        pltpu.sync_copy(agg_sh.at[pl.ds(base, rows_per_sub)],
                        out_hbm.at[c].at[pl.ds(base, rows_per_sub)])

    return k(h_split, src3, dst3)


def kernel(x, edge_index, batch, node_w, node_b, gin_w1, gin_b1, gin_w2,
           gin_b2, eps, out_w1, out_b1, out_w2, out_b2):
    n, _ = x.shape
    e = edge_index.shape[1]
    num_layers = gin_w1.shape[0]

    x_pad = jnp.pad(x, ((0, N_PAD - n), (0, 0)))
    batch_pad = jnp.pad(batch, (0, N_PAD - n), constant_values=16)
    batch3 = batch_pad.reshape(N_PAD // ROW_BLK, 1, ROW_BLK)

    per_sub = -(-e // (SC_SUBCORES * EDGE_BLK)) * EDGE_BLK
    e_pad = per_sub * SC_SUBCORES
    src = jnp.pad(edge_index[0], (0, e_pad - e), constant_values=0)
    dst = jnp.pad(edge_index[1], (0, e_pad - e), constant_values=n)
    src3 = src.reshape(SC_SUBCORES, per_sub // EDGE_BLK, EDGE_BLK)
    dst3 = dst.reshape(SC_SUBCORES, per_sub // EDGE_BLK, EDGE_BLK)

    h = _node_proj(x_pad, node_w, node_b)
    for i in range(num_layers):
        agg = _seg_sum(h, src3, dst3, per_sub // EDGE_BLK)
        scale_row = jnp.full((1, 256), 1.0, jnp.float32) + eps[i]
        h = _gin_mlp(h, agg, scale_row, gin_w1[i], gin_b1[i], gin_w2[i],
                     gin_b2[i])
    return _pool_out(h, batch3, out_w1, out_b1, out_w2, out_b2)


# speedup vs baseline: 2.1521x; 2.1521x over previous
import functools

import jax
import jax.numpy as jnp
from jax import lax
from jax.experimental import pallas as pl
from jax.experimental.pallas import tpu as pltpu
from jax.experimental.pallas import tpu_sc as plsc

N_PAD = 10240
ROW_BLK = 512
SC_CORES = 2
SC_SUBCORES = 16
N_HALF = N_PAD // 2
ACC_ROWS = 5248
EDGE_BLK = 64
BLK_CHUNK = 10
RING = 2


def _node_body(x_ref, w_ref, b_ref, out_ref):
    z = jnp.dot(x_ref[...], w_ref[...], preferred_element_type=jnp.float32,
                precision=lax.Precision.HIGHEST)
    out_ref[...] = jnp.maximum(z + b_ref[...], 0.0)


def _node_proj(x_pad, node_w, node_b):
    return pl.pallas_call(
        _node_body,
        grid=(N_PAD // ROW_BLK,),
        in_specs=[
            pl.BlockSpec((ROW_BLK, 128), lambda i: (i, 0)),
            pl.BlockSpec((128, 256), lambda i: (0, 0)),
            pl.BlockSpec((1, 256), lambda i: (0, 0)),
        ],
        out_specs=pl.BlockSpec((ROW_BLK, 256), lambda i: (i, 0)),
        out_shape=jax.ShapeDtypeStruct((N_PAD, 256), jnp.float32),
    )(x_pad, node_w, node_b.reshape(1, 256))


def _gin_body(h_ref, a_ref, s_ref, w1_ref, b1_ref, w2_ref, b2_ref, out_ref):
    z = h_ref[...] * s_ref[...] + a_ref[...]
    z = jnp.dot(z, w1_ref[...], preferred_element_type=jnp.float32,
                precision=lax.Precision.HIGHEST)
    z = jnp.maximum(z + b1_ref[...], 0.0)
    z = jnp.dot(z, w2_ref[...], preferred_element_type=jnp.float32,
                precision=lax.Precision.HIGHEST)
    out_ref[...] = jnp.maximum(z + b2_ref[...], 0.0)


def _gin_mlp(h, agg, scale_row, w1, b1, w2, b2):
    return pl.pallas_call(
        _gin_body,
        grid=(N_PAD // ROW_BLK,),
        in_specs=[
            pl.BlockSpec((ROW_BLK, 256), lambda i: (i, 0)),
            pl.BlockSpec((ROW_BLK, 256), lambda i: (i, 0)),
            pl.BlockSpec((1, 256), lambda i: (0, 0)),
            pl.BlockSpec((256, 256), lambda i: (0, 0)),
            pl.BlockSpec((1, 256), lambda i: (0, 0)),
            pl.BlockSpec((256, 256), lambda i: (0, 0)),
            pl.BlockSpec((1, 256), lambda i: (0, 0)),
        ],
        out_specs=pl.BlockSpec((ROW_BLK, 256), lambda i: (i, 0)),
        out_shape=jax.ShapeDtypeStruct((N_PAD, 256), jnp.float32),
    )(h, agg, scale_row, w1, b1.reshape(1, 256), w2, b2.reshape(1, 256))


def _pool_body(h_ref, batch_ref, w1_ref, b1_ref, w2_ref, b2_ref, out_ref,
               sums_ref, counts_ref):
    i = pl.program_id(0)
    nblk = pl.num_programs(0)

    @pl.when(i == 0)
    def _():
        sums_ref[...] = jnp.zeros_like(sums_ref)
        counts_ref[...] = jnp.zeros_like(counts_ref)

    b = batch_ref[0, 0, :]
    gids = lax.broadcasted_iota(jnp.int32, (16, ROW_BLK), 0)
    mask = (b[None, :] == gids).astype(jnp.float32)
    sums_ref[...] += jnp.dot(mask, h_ref[...],
                             preferred_element_type=jnp.float32,
                             precision=lax.Precision.HIGHEST)
    counts_ref[...] += jnp.sum(mask, axis=1, keepdims=True)

    @pl.when(i == nblk - 1)
    def _():
        cnt = jnp.maximum(counts_ref[...][:, :1], 1.0)
        hg = sums_ref[...] / cnt
        y = jnp.dot(hg, w1_ref[...], preferred_element_type=jnp.float32,
                    precision=lax.Precision.HIGHEST)
        y = jnp.maximum(y + b1_ref[...], 0.0)
        y = jnp.dot(y, w2_ref[...], preferred_element_type=jnp.float32,
                    precision=lax.Precision.HIGHEST)
        out_ref[...] = y + b2_ref[...]


def _pool_out(h, batch3, out_w1, out_b1, out_w2, out_b2):
    return pl.pallas_call(
        _pool_body,
        grid=(N_PAD // ROW_BLK,),
        in_specs=[
            pl.BlockSpec((ROW_BLK, 256), lambda i: (i, 0)),
            pl.BlockSpec((1, 1, ROW_BLK), lambda i: (i, 0, 0)),
            pl.BlockSpec((256, 256), lambda i: (0, 0)),
            pl.BlockSpec((1, 256), lambda i: (0, 0)),
            pl.BlockSpec((256, 512), lambda i: (0, 0)),
            pl.BlockSpec((1, 512), lambda i: (0, 0)),
        ],
        out_specs=pl.BlockSpec((16, 512), lambda i: (0, 0)),
        out_shape=jax.ShapeDtypeStruct((16, 512), jnp.float32),
        scratch_shapes=[
            pltpu.VMEM((16, 256), jnp.float32),
            pltpu.VMEM((16, 128), jnp.float32),
        ],
    )(h, batch3, out_w1, out_b1.reshape(1, 256), out_w2,
      out_b2.reshape(1, 512))


def _seg_sum(h3, src3, dst4, chunks_per_sub):
    rows_per_sub = ACC_ROWS // SC_SUBCORES
    mesh = plsc.VectorSubcoreMesh(core_axis_name="c", subcore_axis_name="s")

    @functools.partial(
        pl.kernel,
        out_type=jax.ShapeDtypeStruct((2, ACC_ROWS, 2, 128), jnp.float32),
        mesh=mesh,
        scratch_types=[
            pltpu.VMEM((BLK_CHUNK, EDGE_BLK), jnp.int32),
            pltpu.VMEM((BLK_CHUNK, EDGE_BLK), jnp.int32),
        ]
        + [pltpu.VMEM((EDGE_BLK, 2, 128), jnp.float32)] * RING
        + [pltpu.VMEM_SHARED((ACC_ROWS, 2, 128), jnp.float32)]
        + [pltpu.SemaphoreType.DMA] * (2 * RING),
    )
    def k(h_hbm, src_hbm, dst_hbm, out_hbm, src_v, dst_v, *rest):
        bufs = rest[:RING]
        agg_sh = rest[RING]
        gsem = rest[RING + 1:RING + 1 + RING]
        ssem = rest[RING + 1 + RING:]
        c = lax.axis_index("c")
        s = lax.axis_index("s")

        @pl.loop(0, EDGE_BLK)
        def _(r):
            @pl.loop(0, 2)
            def _(hh):
                @pl.loop(0, 128, step=16)
                def _(cg):
                    bufs[0][r, hh, pl.ds(cg, 16)] = jnp.zeros((16,),
                                                              jnp.float32)

        @pl.loop(0, 5)
        def _(i):
            pltpu.sync_copy(
                bufs[0], agg_sh.at[pl.ds(s * rows_per_sub + i * EDGE_BLK,
                                         EDGE_BLK)])
        pltpu.sync_copy(bufs[0].at[pl.ds(0, 8)],
                        agg_sh.at[pl.ds(s * rows_per_sub + 5 * EDGE_BLK, 8)])
        plsc.subcore_barrier()

        @pl.loop(0, chunks_per_sub)
        def _(ch):
            pltpu.sync_copy(src_hbm.at[s].at[ch], src_v)
            pltpu.sync_copy(dst_hbm.at[c].at[s].at[ch], dst_v)
            g = [
                pltpu.async_copy(h_hbm.at[src_v.at[r]], bufs[r], gsem[r])
                for r in range(RING)
            ]
            scat = [None] * RING
            for j in range(BLK_CHUNK):
                b = j % RING
                g[b].wait()
                scat[b] = pltpu.async_copy(bufs[b], agg_sh.at[dst_v.at[j]],
                                           ssem[b], add=True)
                if j + RING < BLK_CHUNK:
                    scat[b].wait()
                    g[b] = pltpu.async_copy(h_hbm.at[src_v.at[j + RING]],
                                            bufs[b], gsem[b])
            for j in range(BLK_CHUNK - RING, BLK_CHUNK):
                scat[j % RING].wait()

        plsc.subcore_barrier()

        base = s * rows_per_sub
        pltpu.sync_copy(agg_sh.at[pl.ds(base, rows_per_sub)],
                        out_hbm.at[c].at[pl.ds(base, rows_per_sub)])

    return k(h3, src3, dst4)


def kernel(x, edge_index, batch, node_w, node_b, gin_w1, gin_b1, gin_w2,
           gin_b2, eps, out_w1, out_b1, out_w2, out_b2):
    n, _ = x.shape
    e = edge_index.shape[1]
    num_layers = gin_w1.shape[0]

    x_pad = jnp.pad(x, ((0, N_PAD - n), (0, 0)))
    batch_pad = jnp.pad(batch, (0, N_PAD - n), constant_values=16)
    batch3 = batch_pad.reshape(N_PAD // ROW_BLK, 1, ROW_BLK)

    chunk_edges = BLK_CHUNK * EDGE_BLK
    per_sub = -(-e // (SC_SUBCORES * chunk_edges)) * chunk_edges
    e_pad = per_sub * SC_SUBCORES
    chunks = per_sub // chunk_edges
    src = jnp.pad(edge_index[0], (0, e_pad - e), constant_values=0)
    dst = jnp.pad(edge_index[1], (0, e_pad - e), constant_values=-1)
    src3 = src.reshape(SC_SUBCORES, chunks, BLK_CHUNK, EDGE_BLK)
    dst_t = dst.reshape(SC_SUBCORES, chunks, BLK_CHUNK, EDGE_BLK)
    junk = (N_HALF + jnp.arange(SC_SUBCORES, dtype=jnp.int32)
            ).reshape(SC_SUBCORES, 1, 1, 1)
    in0 = (dst_t >= 0) & (dst_t < N_HALF)
    in1 = dst_t >= N_HALF
    dst4 = jnp.stack([
        jnp.where(in0, dst_t, junk),
        jnp.where(in1, dst_t - N_HALF, junk),
    ])

    h = _node_proj(x_pad, node_w, node_b)
    for i in range(num_layers):
        agg4 = _seg_sum(h.reshape(N_PAD, 2, 128), src3, dst4, chunks)
        agg = agg4[:, :N_HALF].reshape(N_PAD, 256)
        scale_row = jnp.full((1, 256), 1.0, jnp.float32) + eps[i]
        h = _gin_mlp(h, agg, scale_row, gin_w1[i], gin_b1[i], gin_w2[i],
                     gin_b2[i])
    return _pool_out(h, batch3, out_w1, out_b1, out_w2, out_b2)


# speedup vs baseline: 2.1532x; 1.0005x over previous
import functools

import jax
import jax.numpy as jnp
from jax import lax
from jax.experimental import pallas as pl
from jax.experimental.pallas import tpu as pltpu
from jax.experimental.pallas import tpu_sc as plsc

N_PAD = 10240
ROW_BLK = 512
SC_CORES = 2
SC_SUBCORES = 16
N_HALF = N_PAD // 2
ACC_ROWS = 5248
EDGE_BLK = 64
BLK_CHUNK = 10
RING = 2


def _node_body(x_ref, w_ref, b_ref, out_ref):
    z = jnp.dot(x_ref[...], w_ref[...], preferred_element_type=jnp.float32,
                precision=lax.Precision.HIGHEST)
    out_ref[...] = jnp.maximum(z + b_ref[...], 0.0)


def _node_proj(x_pad, node_w, node_b):
    return pl.pallas_call(
        _node_body,
        grid=(N_PAD // ROW_BLK,),
        in_specs=[
            pl.BlockSpec((ROW_BLK, 128), lambda i: (i, 0)),
            pl.BlockSpec((128, 256), lambda i: (0, 0)),
            pl.BlockSpec((1, 256), lambda i: (0, 0)),
        ],
        out_specs=pl.BlockSpec((ROW_BLK, 256), lambda i: (i, 0)),
        out_shape=jax.ShapeDtypeStruct((N_PAD, 256), jnp.float32),
    )(x_pad, node_w, node_b.reshape(1, 256))


def _gin_body(h_ref, a_ref, s_ref, w1_ref, b1_ref, w2_ref, b2_ref, out_ref):
    z = h_ref[...] * s_ref[...] + a_ref[...]
    z = jnp.dot(z, w1_ref[...], preferred_element_type=jnp.float32,
                precision=lax.Precision.HIGHEST)
    z = jnp.maximum(z + b1_ref[...], 0.0)
    z = jnp.dot(z, w2_ref[...], preferred_element_type=jnp.float32,
                precision=lax.Precision.HIGHEST)
    out_ref[...] = jnp.maximum(z + b2_ref[...], 0.0)


def _gin_mlp(h, agg, scale_row, w1, b1, w2, b2):
    return pl.pallas_call(
        _gin_body,
        grid=(N_PAD // ROW_BLK,),
        in_specs=[
            pl.BlockSpec((ROW_BLK, 256), lambda i: (i, 0)),
            pl.BlockSpec((ROW_BLK, 256), lambda i: (i, 0)),
            pl.BlockSpec((1, 256), lambda i: (0, 0)),
            pl.BlockSpec((256, 256), lambda i: (0, 0)),
            pl.BlockSpec((1, 256), lambda i: (0, 0)),
            pl.BlockSpec((256, 256), lambda i: (0, 0)),
            pl.BlockSpec((1, 256), lambda i: (0, 0)),
        ],
        out_specs=pl.BlockSpec((ROW_BLK, 256), lambda i: (i, 0)),
        out_shape=jax.ShapeDtypeStruct((N_PAD, 256), jnp.float32),
    )(h, agg, scale_row, w1, b1.reshape(1, 256), w2, b2.reshape(1, 256))


def _pool_body(h_ref, batch_ref, w1_ref, b1_ref, w2_ref, b2_ref, out_ref,
               sums_ref, counts_ref):
    i = pl.program_id(0)
    nblk = pl.num_programs(0)

    @pl.when(i == 0)
    def _():
        sums_ref[...] = jnp.zeros_like(sums_ref)
        counts_ref[...] = jnp.zeros_like(counts_ref)

    b = batch_ref[0, 0, :]
    gids = lax.broadcasted_iota(jnp.int32, (16, ROW_BLK), 0)
    mask = (b[None, :] == gids).astype(jnp.float32)
    sums_ref[...] += jnp.dot(mask, h_ref[...],
                             preferred_element_type=jnp.float32,
                             precision=lax.Precision.HIGHEST)
    counts_ref[...] += jnp.sum(mask, axis=1, keepdims=True)

    @pl.when(i == nblk - 1)
    def _():
        cnt = jnp.maximum(counts_ref[...][:, :1], 1.0)
        hg = sums_ref[...] / cnt
        y = jnp.dot(hg, w1_ref[...], preferred_element_type=jnp.float32,
                    precision=lax.Precision.HIGHEST)
        y = jnp.maximum(y + b1_ref[...], 0.0)
        y = jnp.dot(y, w2_ref[...], preferred_element_type=jnp.float32,
                    precision=lax.Precision.HIGHEST)
        out_ref[...] = y + b2_ref[...]


def _pool_out(h, batch3, out_w1, out_b1, out_w2, out_b2):
    return pl.pallas_call(
        _pool_body,
        grid=(N_PAD // ROW_BLK,),
        in_specs=[
            pl.BlockSpec((ROW_BLK, 256), lambda i: (i, 0)),
            pl.BlockSpec((1, 1, ROW_BLK), lambda i: (i, 0, 0)),
            pl.BlockSpec((256, 256), lambda i: (0, 0)),
            pl.BlockSpec((1, 256), lambda i: (0, 0)),
            pl.BlockSpec((256, 512), lambda i: (0, 0)),
            pl.BlockSpec((1, 512), lambda i: (0, 0)),
        ],
        out_specs=pl.BlockSpec((16, 512), lambda i: (0, 0)),
        out_shape=jax.ShapeDtypeStruct((16, 512), jnp.float32),
        scratch_shapes=[
            pltpu.VMEM((16, 256), jnp.float32),
            pltpu.VMEM((16, 128), jnp.float32),
        ],
    )(h, batch3, out_w1, out_b1.reshape(1, 256), out_w2,
      out_b2.reshape(1, 512))


def _seg_sum(h3, src3, dst4, chunks_per_sub):
    rows_per_sub = ACC_ROWS // SC_SUBCORES
    mesh = plsc.VectorSubcoreMesh(core_axis_name="c", subcore_axis_name="s")

    @functools.partial(
        pl.kernel,
        out_type=jax.ShapeDtypeStruct((2, ACC_ROWS, 2, 128), jnp.float32),
        mesh=mesh,
        scratch_types=[
            pltpu.VMEM((BLK_CHUNK, EDGE_BLK), jnp.int32),
            pltpu.VMEM((BLK_CHUNK, EDGE_BLK), jnp.int32),
        ]
        + [pltpu.VMEM((EDGE_BLK, 2, 128), jnp.float32)] * RING
        + [pltpu.VMEM_SHARED((ACC_ROWS, 2, 128), jnp.float32)]
        + [pltpu.SemaphoreType.DMA] * (2 * RING),
    )
    def k(h_hbm, src_hbm, dst_hbm, out_hbm, src_v, dst_v, *rest):
        bufs = rest[:RING]
        agg_sh = rest[RING]
        gsem = rest[RING + 1:RING + 1 + RING]
        ssem = rest[RING + 1 + RING:]
        c = lax.axis_index("c")
        s = lax.axis_index("s")

        @pl.loop(0, EDGE_BLK)
        def _(r):
            @pl.loop(0, 2)
            def _(hh):
                @pl.loop(0, 128, step=16)
                def _(cg):
                    bufs[0][r, hh, pl.ds(cg, 16)] = jnp.zeros((16,),
                                                              jnp.float32)

        @pl.loop(0, 5)
        def _(i):
            pltpu.sync_copy(
                bufs[0], agg_sh.at[pl.ds(s * rows_per_sub + i * EDGE_BLK,
                                         EDGE_BLK)])
        pltpu.sync_copy(bufs[0].at[pl.ds(0, 8)],
                        agg_sh.at[pl.ds(s * rows_per_sub + 5 * EDGE_BLK, 8)])
        plsc.subcore_barrier()

        @pl.loop(0, chunks_per_sub)
        def _(ch):
            pltpu.sync_copy(src_hbm.at[s].at[ch], src_v)
            pltpu.sync_copy(dst_hbm.at[c].at[s].at[ch], dst_v)
            g = [
                pltpu.async_copy(h_hbm.at[src_v.at[r]], bufs[r], gsem[r])
                for r in range(RING)
            ]
            scat = [None] * RING
            for j in range(BLK_CHUNK):
                b = j % RING
                g[b].wait()
                scat[b] = pltpu.async_copy(bufs[b], agg_sh.at[dst_v.at[j]],
                                           ssem[b], add=True)
                if j + RING < BLK_CHUNK:
                    scat[b].wait()
                    g[b] = pltpu.async_copy(h_hbm.at[src_v.at[j + RING]],
                                            bufs[b], gsem[b])
            for j in range(BLK_CHUNK - RING, BLK_CHUNK):
                scat[j % RING].wait()

        plsc.subcore_barrier()

        base = s * rows_per_sub
        pltpu.sync_copy(agg_sh.at[pl.ds(base, rows_per_sub)],
                        out_hbm.at[c].at[pl.ds(base, rows_per_sub)])

    return k(h3, src3, dst4)


def kernel(x, edge_index, batch, node_w, node_b, gin_w1, gin_b1, gin_w2,
           gin_b2, eps, out_w1, out_b1, out_w2, out_b2):
    n, _ = x.shape
    e = edge_index.shape[1]
    num_layers = gin_w1.shape[0]

    x_pad = jnp.pad(x, ((0, N_PAD - n), (0, 0)))
    batch_pad = jnp.pad(batch, (0, N_PAD - n), constant_values=16)
    batch3 = batch_pad.reshape(N_PAD // ROW_BLK, 1, ROW_BLK)

    chunk_edges = BLK_CHUNK * EDGE_BLK
    per_sub = -(-e // (SC_SUBCORES * chunk_edges)) * chunk_edges
    e_pad = per_sub * SC_SUBCORES
    chunks = per_sub // chunk_edges
    src = jnp.pad(edge_index[0], (0, e_pad - e), constant_values=0)
    dst = jnp.pad(edge_index[1], (0, e_pad - e), constant_values=-1)
    src3 = src.reshape(SC_SUBCORES, chunks, BLK_CHUNK, EDGE_BLK)
    dst_t = dst.reshape(SC_SUBCORES, chunks, BLK_CHUNK, EDGE_BLK)
    junk = (N_HALF + jnp.arange(e_pad, dtype=jnp.int32) % (ACC_ROWS - N_HALF)
            ).reshape(SC_SUBCORES, chunks, BLK_CHUNK, EDGE_BLK)
    in0 = (dst_t >= 0) & (dst_t < N_HALF)
    in1 = dst_t >= N_HALF
    dst4 = jnp.stack([
        jnp.where(in0, dst_t, junk),
        jnp.where(in1, dst_t - N_HALF, junk),
    ])

    h = _node_proj(x_pad, node_w, node_b)
    for i in range(num_layers):
        agg4 = _seg_sum(h.reshape(N_PAD, 2, 128), src3, dst4, chunks)
        agg = agg4[:, :N_HALF].reshape(N_PAD, 256)
        scale_row = jnp.full((1, 256), 1.0, jnp.float32) + eps[i]
        h = _gin_mlp(h, agg, scale_row, gin_w1[i], gin_b1[i], gin_w2[i],
                     gin_b2[i])
    return _pool_out(h, batch3, out_w1, out_b1, out_w2, out_b2)


# speedup vs baseline: 3.1390x; 1.4578x over previous
import functools

import jax
import jax.numpy as jnp
from jax import lax
from jax.experimental import pallas as pl
from jax.experimental.pallas import tpu as pltpu
from jax.experimental.pallas import tpu_sc as plsc

N_PAD = 10240
ROW_BLK = 512
SC_CORES = 2
SC_SUBCORES = 16
EDGE_BLK = 64


def _node_body(x_ref, w_ref, b_ref, out_ref):
    z = jnp.dot(x_ref[...], w_ref[...], preferred_element_type=jnp.float32,
                precision=lax.Precision.HIGHEST)
    z = jnp.maximum(z + b_ref[...], 0.0)
    out_ref[0] = z[:, :128]
    out_ref[1] = z[:, 128:]


def _node_proj(x_pad, node_w, node_b):
    grid = (N_PAD // ROW_BLK,)
    return pl.pallas_call(
        _node_body,
        grid=grid,
        in_specs=[
            pl.BlockSpec((ROW_BLK, 128), lambda i: (i, 0)),
            pl.BlockSpec((128, 256), lambda i: (0, 0)),
            pl.BlockSpec((1, 256), lambda i: (0, 0)),
        ],
        out_specs=pl.BlockSpec((2, ROW_BLK, 128), lambda i: (0, i, 0)),
        out_shape=jax.ShapeDtypeStruct((2, N_PAD, 128), jnp.float32),
    )(x_pad, node_w, node_b.reshape(1, 256))


def _gin_body(h_ref, a_ref, s_ref, w1_ref, b1_ref, w2_ref, b2_ref, out_ref):
    h = jnp.concatenate([h_ref[0], h_ref[1]], axis=1)
    agg = jnp.concatenate([a_ref[0], a_ref[1]], axis=1)
    z = h * s_ref[...] + agg
    z = jnp.dot(z, w1_ref[...], preferred_element_type=jnp.float32,
                precision=lax.Precision.HIGHEST)
    z = jnp.maximum(z + b1_ref[...], 0.0)
    z = jnp.dot(z, w2_ref[...], preferred_element_type=jnp.float32,
                precision=lax.Precision.HIGHEST)
    z = jnp.maximum(z + b2_ref[...], 0.0)
    out_ref[0] = z[:, :128]
    out_ref[1] = z[:, 128:]


def _gin_mlp(h_split, agg_split, scale_row, w1, b1, w2, b2):
    grid = (N_PAD // ROW_BLK,)
    return pl.pallas_call(
        _gin_body,
        grid=grid,
        in_specs=[
            pl.BlockSpec((2, ROW_BLK, 128), lambda i: (0, i, 0)),
            pl.BlockSpec((2, ROW_BLK, 128), lambda i: (0, i, 0)),
            pl.BlockSpec((1, 256), lambda i: (0, 0)),
            pl.BlockSpec((256, 256), lambda i: (0, 0)),
            pl.BlockSpec((1, 256), lambda i: (0, 0)),
            pl.BlockSpec((256, 256), lambda i: (0, 0)),
            pl.BlockSpec((1, 256), lambda i: (0, 0)),
        ],
        out_specs=pl.BlockSpec((2, ROW_BLK, 128), lambda i: (0, i, 0)),
        out_shape=jax.ShapeDtypeStruct((2, N_PAD, 128), jnp.float32),
    )(h_split, agg_split, scale_row, w1, b1.reshape(1, 256), w2,
      b2.reshape(1, 256))


def _pool_body(h_ref, batch_ref, w1_ref, b1_ref, w2_ref, b2_ref, out_ref,
               sums_ref, counts_ref):
    i = pl.program_id(0)
    nblk = pl.num_programs(0)

    @pl.when(i == 0)
    def _():
        sums_ref[...] = jnp.zeros_like(sums_ref)
        counts_ref[...] = jnp.zeros_like(counts_ref)

    h = jnp.concatenate([h_ref[0], h_ref[1]], axis=1)
    b = batch_ref[0, 0, :]
    gids = lax.broadcasted_iota(jnp.int32, (16, ROW_BLK), 0)
    mask = (b[None, :] == gids).astype(jnp.float32)
    sums_ref[...] += jnp.dot(mask, h, preferred_element_type=jnp.float32,
                             precision=lax.Precision.HIGHEST)
    counts_ref[...] += jnp.sum(mask, axis=1, keepdims=True)

    @pl.when(i == nblk - 1)
    def _():
        cnt = jnp.maximum(counts_ref[...][:, :1], 1.0)
        hg = sums_ref[...] / cnt
        y = jnp.dot(hg, w1_ref[...], preferred_element_type=jnp.float32,
                    precision=lax.Precision.HIGHEST)
        y = jnp.maximum(y + b1_ref[...], 0.0)
        y = jnp.dot(y, w2_ref[...], preferred_element_type=jnp.float32,
                    precision=lax.Precision.HIGHEST)
        out_ref[...] = y + b2_ref[...]


def _pool_out(h_split, batch3, out_w1, out_b1, out_w2, out_b2):
    grid = (N_PAD // ROW_BLK,)
    return pl.pallas_call(
        _pool_body,
        grid=grid,
        in_specs=[
            pl.BlockSpec((2, ROW_BLK, 128), lambda i: (0, i, 0)),
            pl.BlockSpec((1, 1, ROW_BLK), lambda i: (i, 0, 0)),
            pl.BlockSpec((256, 256), lambda i: (0, 0)),
            pl.BlockSpec((1, 256), lambda i: (0, 0)),
            pl.BlockSpec((256, 512), lambda i: (0, 0)),
            pl.BlockSpec((1, 512), lambda i: (0, 0)),
        ],
        out_specs=pl.BlockSpec((16, 512), lambda i: (0, 0)),
        out_shape=jax.ShapeDtypeStruct((16, 512), jnp.float32),
        scratch_shapes=[
            pltpu.VMEM((16, 256), jnp.float32),
            pltpu.VMEM((16, 128), jnp.float32),
        ],
    )(h_split, batch3, out_w1, out_b1.reshape(1, 256), out_w2,
      out_b2.reshape(1, 512))


BLK_CHUNK = 10
RING = 5


def _seg_sum(h_split, src3, dst3, chunks_per_sub):
    rows_per_sub = N_PAD // SC_SUBCORES
    mesh = plsc.VectorSubcoreMesh(core_axis_name="c", subcore_axis_name="s")

    @functools.partial(
        pl.kernel,
        out_type=jax.ShapeDtypeStruct((2, N_PAD, 128), jnp.float32),
        mesh=mesh,
        scratch_types=[
            pltpu.VMEM((BLK_CHUNK, EDGE_BLK), jnp.int32),
            pltpu.VMEM((BLK_CHUNK, EDGE_BLK), jnp.int32),
        ]
        + [pltpu.VMEM((EDGE_BLK, 128), jnp.float32)] * RING
        + [pltpu.VMEM_SHARED((N_PAD, 128), jnp.float32)]
        + [pltpu.SemaphoreType.DMA] * (2 * RING),
    )
    def k(h_hbm, src_hbm, dst_hbm, out_hbm, src_v, dst_v, *rest):
        bufs = rest[:RING]
        agg_sh = rest[RING]
        gsem = rest[RING + 1:RING + 1 + RING]
        ssem = rest[RING + 1 + RING:]
        c = lax.axis_index("c")
        s = lax.axis_index("s")
        table = h_hbm.at[c]

        @pl.loop(0, EDGE_BLK)
        def _(r):
            @pl.loop(0, 128, step=16)
            def _(cg):
                bufs[0][r, pl.ds(cg, 16)] = jnp.zeros((16,), jnp.float32)

        @pl.loop(0, rows_per_sub // EDGE_BLK)
        def _(i):
            pltpu.sync_copy(
                bufs[0], agg_sh.at[pl.ds(s * rows_per_sub + i * EDGE_BLK,
                                         EDGE_BLK)])
        plsc.subcore_barrier()

        @pl.loop(0, chunks_per_sub)
        def _(ch):
            pltpu.sync_copy(src_hbm.at[s].at[ch], src_v)
            pltpu.sync_copy(dst_hbm.at[s].at[ch], dst_v)
            g = [
                pltpu.async_copy(table.at[src_v.at[r]], bufs[r], gsem[r])
                for r in range(RING)
            ]
            scat = [None] * RING
            for j in range(BLK_CHUNK):
                b = j % RING
                g[b].wait()
                scat[b] = pltpu.async_copy(bufs[b], agg_sh.at[dst_v.at[j]],
                                           ssem[b], add=True)
                if j + RING < BLK_CHUNK:
                    scat[b].wait()
                    g[b] = pltpu.async_copy(table.at[src_v.at[j + RING]],
                                            bufs[b], gsem[b])
            for j in range(BLK_CHUNK - RING, BLK_CHUNK):
                scat[j % RING].wait()

        plsc.subcore_barrier()

        base = s * rows_per_sub
        pltpu.sync_copy(agg_sh.at[pl.ds(base, rows_per_sub)],
                        out_hbm.at[c].at[pl.ds(base, rows_per_sub)])

    return k(h_split, src3, dst3)


def kernel(x, edge_index, batch, node_w, node_b, gin_w1, gin_b1, gin_w2,
           gin_b2, eps, out_w1, out_b1, out_w2, out_b2):
    n, _ = x.shape
    e = edge_index.shape[1]
    num_layers = gin_w1.shape[0]

    x_pad = jnp.pad(x, ((0, N_PAD - n), (0, 0)))
    batch_pad = jnp.pad(batch, (0, N_PAD - n), constant_values=16)
    batch3 = batch_pad.reshape(N_PAD // ROW_BLK, 1, ROW_BLK)

    chunk_edges = BLK_CHUNK * EDGE_BLK
    per_sub = -(-e // (SC_SUBCORES * chunk_edges)) * chunk_edges
    e_pad = per_sub * SC_SUBCORES
    src = jnp.pad(edge_index[0], (0, e_pad - e), constant_values=0)
    dst = jnp.pad(edge_index[1], (0, e_pad - e), constant_values=n)
    src3 = src.reshape(SC_SUBCORES, per_sub // chunk_edges, BLK_CHUNK,
                       EDGE_BLK)
    dst3 = dst.reshape(SC_SUBCORES, per_sub // chunk_edges, BLK_CHUNK,
                       EDGE_BLK)

    h = _node_proj(x_pad, node_w, node_b)
    for i in range(num_layers):
        agg = _seg_sum(h, src3, dst3, per_sub // chunk_edges)
        scale_row = jnp.full((1, 256), 1.0, jnp.float32) + eps[i]
        h = _gin_mlp(h, agg, scale_row, gin_w1[i], gin_b1[i], gin_w2[i],
                     gin_b2[i])
    return _pool_out(h, batch3, out_w1, out_b1, out_w2, out_b2)
